# baseline (speedup 1.0000x reference)
import functools

import jax
import jax.numpy as jnp
from jax import lax
from jax.experimental import pallas as pl
from jax.experimental.pallas import tpu as pltpu
from jax.experimental.pallas import tpu_sc as plsc

_B = 1024
_NC = 2
_NS = 16
_NW = _NC * _NS
_BPT = _B // _NW
_L = 16


def _sc_body(pred_hbm, tgt_hbm, rew_hbm, out_hbm, tgt_v, rew_v, win_v, part_v):
    cid = lax.axis_index("c")
    sid = lax.axis_index("s")
    wid = sid * _NC + cid
    base = wid * _BPT

    pltpu.sync_copy(tgt_hbm.at[pl.ds(base, _BPT)], tgt_v)
    pltpu.sync_copy(rew_hbm.at[pl.ds(base, _BPT)], rew_v)
    lane = lax.iota(jnp.int32, _L)
    acc = jnp.zeros((_L,), jnp.float32)
    for k in range(_BPT // _L):
        rows = (k * _L + lane) * 8 + (lane & 7)
        offs = tgt_v[pl.ds(k * _L, _L)] & 127
        vals = plsc.load_gather(win_v, [rows, offs])
        acc = acc + vals * rew_v[pl.ds(k * _L, _L)]
    part_v[...] = -acc
    pltpu.sync_copy(part_v, out_hbm.at[wid])


_sc_call = functools.partial(
    pl.kernel,
    mesh=plsc.VectorSubcoreMesh(core_axis_name="c", subcore_axis_name="s"),
    out_type=jax.ShapeDtypeStruct((_NW, _L), jnp.float32),
    compiler_params=pltpu.CompilerParams(
        needs_layout_passes=False, skip_device_barrier=True),
    scratch_types=[
        pltpu.VMEM((_BPT,), jnp.int32),
        pltpu.VMEM((_BPT,), jnp.float32),
        pltpu.VMEM((_BPT * 8, 128), jnp.float32),
        pltpu.VMEM((_L,), jnp.float32),
    ],
)(_sc_body)


def kernel(pred, target, reward):
    parts = _sc_call(pred, target.astype(jnp.int32), reward)
    return jnp.sum(parts)


# speedup vs baseline: 14.9125x; 14.9125x over previous
import functools

import jax
import jax.numpy as jnp
from jax import lax
from jax.experimental import pallas as pl
from jax.experimental.pallas import tpu as pltpu
from jax.experimental.pallas import tpu_sc as plsc

_B = 1024
_V = 100000
_NC = 2
_NS = 16
_NW = _NC * _NS
_BPT = _B // _NW
_L = 16


def _sc_body(predt_hbm, tgt_hbm, rew_hbm, out_hbm,
             tgt_v, rew_v, win_v, part_v, sem):
    cid = lax.axis_index("c")
    sid = lax.axis_index("s")
    wid = sid * _NC + cid
    base = wid * _BPT
    b0 = (wid // 4) * 128
    boff = (wid % 4) * _BPT

    pltpu.sync_copy(tgt_hbm.at[pl.ds(base, _BPT)], tgt_v)
    pltpu.sync_copy(rew_hbm.at[pl.ds(base, _BPT)], rew_v)

    tchunks = [tgt_v[pl.ds(k * _L, _L)] for k in range(_BPT // _L)]
    copies = []
    for j in range(_BPT):
        v = tchunks[j // _L][j % _L]
        v0 = pl.multiple_of(v & jnp.int32(~7), 8)
        copies.append(
            pltpu.make_async_copy(
                predt_hbm.at[pl.ds(v0, 8), pl.ds(b0, 128)],
                win_v.at[pl.ds(j * 8, 8)], sem))
    for c in copies:
        c.start()
    for c in copies:
        c.wait()

    lane = lax.iota(jnp.int32, _L)
    acc = jnp.zeros((_L,), jnp.float32)
    for k in range(_BPT // _L):
        js = k * _L + lane
        rows = js * 8 + (tchunks[k] & 7)
        cols = boff + js
        vals = plsc.load_gather(win_v, [rows, cols])
        acc = acc + vals * rew_v[pl.ds(k * _L, _L)]
    part_v[...] = -acc
    pltpu.sync_copy(part_v, out_hbm.at[wid])


_sc_call = functools.partial(
    pl.kernel,
    mesh=plsc.VectorSubcoreMesh(core_axis_name="c", subcore_axis_name="s"),
    out_type=jax.ShapeDtypeStruct((_NW, _L), jnp.float32),
    compiler_params=pltpu.CompilerParams(
        needs_layout_passes=False, skip_device_barrier=True),
    scratch_types=[
        pltpu.VMEM((_BPT,), jnp.int32),
        pltpu.VMEM((_BPT,), jnp.float32),
        pltpu.VMEM((_BPT * 8, 128), jnp.float32),
        pltpu.VMEM((_L,), jnp.float32),
        pltpu.SemaphoreType.DMA,
    ],
)(_sc_body)


def kernel(pred, target, reward):
    parts = _sc_call(pred.T, target.astype(jnp.int32), reward)
    return jnp.sum(parts)


# speedup vs baseline: 15.0502x; 1.0092x over previous
import functools

import jax
import jax.numpy as jnp
from jax import lax
from jax.experimental import pallas as pl
from jax.experimental.pallas import tpu as pltpu
from jax.experimental.pallas import tpu_sc as plsc

_B = 1024
_V = 100000
_NC = 1
_NS = 16
_NW = _NC * _NS
_BPT = _B // _NW
_L = 16


def _sc_body(predt_hbm, tgt_hbm, rew_hbm, out_hbm,
             tgt_v, rew_v, win_v, part_v, sem):
    cid = lax.axis_index("c")
    sid = lax.axis_index("s")
    wid = sid * _NC + cid
    base = wid * _BPT
    b0 = (base // 128) * 128
    boff = base % 128

    pltpu.sync_copy(tgt_hbm.at[pl.ds(base, _BPT)], tgt_v)
    pltpu.sync_copy(rew_hbm.at[pl.ds(base, _BPT)], rew_v)

    tchunks = [tgt_v[pl.ds(k * _L, _L)] for k in range(_BPT // _L)]
    copies = []
    for j in range(_BPT):
        v = tchunks[j // _L][j % _L]
        v0 = pl.multiple_of(v & jnp.int32(~7), 8)
        copies.append(
            pltpu.make_async_copy(
                predt_hbm.at[pl.ds(v0, 8), pl.ds(b0, 128)],
                win_v.at[pl.ds(j * 8, 8)], sem))
    for c in copies:
        c.start()
    for c in copies:
        c.wait()

    lane = lax.iota(jnp.int32, _L)
    acc = jnp.zeros((_L,), jnp.float32)
    for k in range(_BPT // _L):
        js = k * _L + lane
        rows = js * 8 + (tchunks[k] & 7)
        cols = boff + js
        vals = plsc.load_gather(win_v, [rows, cols])
        acc = acc + vals * rew_v[pl.ds(k * _L, _L)]
    part_v[...] = -acc
    pltpu.sync_copy(part_v, out_hbm.at[wid])


_sc_call = functools.partial(
    pl.kernel,
    mesh=plsc.VectorSubcoreMesh(
        core_axis_name="c", subcore_axis_name="s", num_cores=_NC),
    out_type=jax.ShapeDtypeStruct((_NW, _L), jnp.float32),
    compiler_params=pltpu.CompilerParams(
        needs_layout_passes=False, skip_device_barrier=True),
    scratch_types=[
        pltpu.VMEM((_BPT,), jnp.int32),
        pltpu.VMEM((_BPT,), jnp.float32),
        pltpu.VMEM((_BPT * 8, 128), jnp.float32),
        pltpu.VMEM((_L,), jnp.float32),
        pltpu.SemaphoreType.DMA,
    ],
)(_sc_body)


def kernel(pred, target, reward):
    parts = _sc_call(pred.T, target.astype(jnp.int32), reward)
    return jnp.sum(parts)


# speedup vs baseline: 15.0720x; 1.0014x over previous
import functools

import jax
import jax.numpy as jnp
from jax import lax
from jax.experimental import pallas as pl
from jax.experimental.pallas import tpu as pltpu
from jax.experimental.pallas import tpu_sc as plsc

_B = 1024
_V = 100000
_NC = 2
_NS = 16
_NW = _NC * _NS
_BPT = _B // _NW
_L = 16


def _sc_body(predt_hbm, tgt_hbm, rew_hbm, out_hbm,
             tgt_v, rew_v, win_v, part_v, sem):
    cid = lax.axis_index("c")
    sid = lax.axis_index("s")
    wid = sid * _NC + cid
    base = wid * _BPT
    b0 = (wid // 4) * 128
    boff = (wid % 4) * _BPT

    pltpu.sync_copy(tgt_hbm.at[pl.ds(base, _BPT)], tgt_v)
    pltpu.sync_copy(rew_hbm.at[pl.ds(base, _BPT)], rew_v)

    tchunks = [tgt_v[pl.ds(k * _L, _L)] for k in range(_BPT // _L)]
    copies = []
    for j in range(_BPT):
        v = tchunks[j // _L][j % _L]
        v0 = pl.multiple_of(v & jnp.int32(~7), 8)
        copies.append(
            pltpu.make_async_copy(
                predt_hbm.at[pl.ds(v0, 8), pl.ds(b0, 128)],
                win_v.at[pl.ds(j * 8, 8)], sem))
    for c in copies:
        c.start()
    for c in copies:
        c.wait()

    lane = lax.iota(jnp.int32, _L)
    acc = jnp.zeros((_L,), jnp.float32)
    for k in range(_BPT // _L):
        js = k * _L + lane
        rows = js * 8 + (tchunks[k] & 7)
        cols = boff + js
        vals = plsc.load_gather(win_v, [rows, cols])
        acc = acc + vals * rew_v[pl.ds(k * _L, _L)]
    part_v[...] = -acc
    pltpu.sync_copy(part_v, out_hbm.at[wid])


_sc_call = functools.partial(
    pl.kernel,
    mesh=plsc.VectorSubcoreMesh(core_axis_name="c", subcore_axis_name="s"),
    out_type=jax.ShapeDtypeStruct((_NW, _L), jnp.float32),
    compiler_params=pltpu.CompilerParams(
        needs_layout_passes=False, skip_device_barrier=True),
    scratch_types=[
        pltpu.VMEM((_BPT,), jnp.int32),
        pltpu.VMEM((_BPT,), jnp.float32),
        pltpu.VMEM((_BPT * 8, 128), jnp.float32),
        pltpu.VMEM((_L,), jnp.float32),
        pltpu.SemaphoreType.DMA,
    ],
)(_sc_body)


def kernel(pred, target, reward):
    parts = _sc_call(pred.T, target.astype(jnp.int32), reward)
    return jnp.sum(parts)


# speedup vs baseline: 15.5238x; 1.0300x over previous
import functools

import jax
import jax.numpy as jnp
from jax import lax
from jax.experimental import pallas as pl
from jax.experimental.pallas import tpu as pltpu
from jax.experimental.pallas import tpu_sc as plsc

_B = 1024
_V = 100000
_NC = 2
_NS = 16
_NW = _NC * _NS
_BPT = _B // _NW
_L = 16


def _sc_body(w_hbm, tgt_hbm, rew_hbm, out_hbm,
             tgt_v, rew_v, idx_v, win_v, part_v, sem):
    cid = lax.axis_index("c")
    sid = lax.axis_index("s")
    wid = sid * _NC + cid
    base = wid * _BPT
    bblk = wid // 4
    boff = (wid % 4) * _BPT

    pltpu.sync_copy(tgt_hbm.at[pl.ds(base, _BPT)], tgt_v)
    pltpu.sync_copy(rew_hbm.at[pl.ds(base, _BPT)], rew_v)

    tchunks = [tgt_v[pl.ds(k * _L, _L)] for k in range(_BPT // _L)]
    for k in range(_BPT // _L):
        idx_v[pl.ds(k * _L, _L)] = (tchunks[k] & jnp.int32(~7)) + bblk
    pltpu.async_copy(w_hbm.at[idx_v], win_v, sem).wait()

    lane = lax.iota(jnp.int32, _L)
    acc = jnp.zeros((_L,), jnp.float32)
    for k in range(_BPT // _L):
        js = k * _L + lane
        subs = tchunks[k] & 7
        lanes = boff + js
        vals = plsc.load_gather(win_v, [js, subs, lanes])
        acc = acc + vals * rew_v[pl.ds(k * _L, _L)]
    part_v[...] = -acc
    pltpu.sync_copy(part_v, out_hbm.at[wid])


_sc_call = functools.partial(
    pl.kernel,
    mesh=plsc.VectorSubcoreMesh(core_axis_name="c", subcore_axis_name="s"),
    out_type=jax.ShapeDtypeStruct((_NW, _L), jnp.float32),
    compiler_params=pltpu.CompilerParams(
        needs_layout_passes=False, skip_device_barrier=True),
    scratch_types=[
        pltpu.VMEM((_BPT,), jnp.int32),
        pltpu.VMEM((_BPT,), jnp.float32),
        pltpu.VMEM((_BPT,), jnp.int32),
        pltpu.VMEM((_BPT, 8, 128), jnp.float32),
        pltpu.VMEM((_L,), jnp.float32),
        pltpu.SemaphoreType.DMA,
    ],
)(_sc_body)


def kernel(pred, target, reward):
    w = (pred.T.reshape(_V // 8, 8, 8, 128)
         .transpose(0, 2, 1, 3)
         .reshape(_V, 8, 128))
    parts = _sc_call(w, target.astype(jnp.int32), reward)
    return jnp.sum(parts)


# speedup vs baseline: 15.7579x; 1.0151x over previous
import functools

import jax
import jax.numpy as jnp
from jax import lax
from jax.experimental import pallas as pl
from jax.experimental.pallas import tpu as pltpu
from jax.experimental.pallas import tpu_sc as plsc

_B = 1024
_V = 100000
_NC = 2
_NS = 16
_NW = _NC * _NS
_BPT = _B // _NW
_L = 16


def _sc_body(w_hbm, tgt_hbm, rew_hbm, out_hbm,
             tgt_v, rew_v, idx_v, win_v, part_v, sem):
    cid = lax.axis_index("c")
    sid = lax.axis_index("s")
    wid = sid * _NC + cid
    base = wid * _BPT
    bblk = wid // 4
    boff = (wid % 4) * _BPT

    pltpu.sync_copy(tgt_hbm.at[pl.ds(base, _BPT)], tgt_v)

    tchunks = [tgt_v[pl.ds(k * _L, _L)] for k in range(_BPT // _L)]
    for k in range(_BPT // _L):
        idx_v[pl.ds(k * _L, _L)] = (tchunks[k] & jnp.int32(~7)) + bblk
    gather = pltpu.make_async_copy(w_hbm.at[idx_v], win_v, sem)
    gather.start()
    pltpu.sync_copy(rew_hbm.at[pl.ds(base, _BPT)], rew_v)
    gather.wait()

    lane = lax.iota(jnp.int32, _L)
    acc = jnp.zeros((_L,), jnp.float32)
    for k in range(_BPT // _L):
        js = k * _L + lane
        subs = tchunks[k] & 7
        lanes = boff + js
        vals = plsc.load_gather(win_v, [js, subs, lanes])
        acc = acc + vals * rew_v[pl.ds(k * _L, _L)]
    part_v[...] = -acc
    pltpu.sync_copy(part_v, out_hbm.at[wid])


_sc_call = functools.partial(
    pl.kernel,
    mesh=plsc.VectorSubcoreMesh(core_axis_name="c", subcore_axis_name="s"),
    out_type=jax.ShapeDtypeStruct((_NW, _L), jnp.float32),
    compiler_params=pltpu.CompilerParams(
        needs_layout_passes=False, skip_device_barrier=True),
    scratch_types=[
        pltpu.VMEM((_BPT,), jnp.int32),
        pltpu.VMEM((_BPT,), jnp.float32),
        pltpu.VMEM((_BPT,), jnp.int32),
        pltpu.VMEM((_BPT, 8, 128), jnp.float32),
        pltpu.VMEM((_L,), jnp.float32),
        pltpu.SemaphoreType.DMA,
    ],
)(_sc_body)


def kernel(pred, target, reward):
    w = (pred.T.reshape(_V // 8, 8, 8, 128)
         .transpose(0, 2, 1, 3)
         .reshape(_V, 8, 128))
    parts = _sc_call(w, target.astype(jnp.int32), reward)
    return jnp.sum(parts)
